# baseline (speedup 1.0000x reference)
import jax
import jax.numpy as jnp
from jax.experimental import pallas as pl
from jax.experimental.pallas import tpu as pltpu

N_ROWS = 16384
N_COLS = 16320


def _gather_row_body(idx_ref, hbm_ref, out_ref, sem):
    row = idx_ref[0]
    copy = pltpu.make_async_copy(hbm_ref.at[pl.ds(row, 1), :], out_ref, sem)
    copy.start()
    copy.wait()


_gather_row = pl.pallas_call(
    _gather_row_body,
    grid_spec=pltpu.PrefetchScalarGridSpec(
        num_scalar_prefetch=1,
        grid=(1,),
        in_specs=[pl.BlockSpec(memory_space=pl.ANY)],
        out_specs=pl.BlockSpec(memory_space=pl.ANY),
        scratch_shapes=[pltpu.SemaphoreType.DMA],
    ),
    out_shape=jax.ShapeDtypeStruct((1, N_ROWS), jnp.float32),
)


def kernel(state, index):
    idx = jnp.asarray(index, jnp.int32).reshape(1)
    return _gather_row(idx, state.T).reshape(N_ROWS)


# speedup vs baseline: 1.7377x; 1.7377x over previous
import jax
import jax.numpy as jnp
from jax.experimental import pallas as pl
from jax.experimental.pallas import tpu as pltpu

N_ROWS = 16384
N_COLS = 16320


N_SPLIT = 4
SPLIT_C = N_ROWS // N_SPLIT


def _gather_row_body(idx_ref, hbm_ref, out_ref, sems):
    row = idx_ref[0]
    copies = [
        pltpu.make_async_copy(
            hbm_ref.at[pl.ds(row, 1), pl.ds(k * SPLIT_C, SPLIT_C)],
            out_ref.at[:, pl.ds(k * SPLIT_C, SPLIT_C)],
            sems.at[k],
        )
        for k in range(N_SPLIT)
    ]
    for c in copies:
        c.start()
    for c in copies:
        c.wait()


_gather_row = pl.pallas_call(
    _gather_row_body,
    grid_spec=pltpu.PrefetchScalarGridSpec(
        num_scalar_prefetch=1,
        grid=(1,),
        in_specs=[pl.BlockSpec(memory_space=pl.ANY)],
        out_specs=pl.BlockSpec((1, N_ROWS), lambda i, idx: (0, 0)),
        scratch_shapes=[pltpu.SemaphoreType.DMA((N_SPLIT,))],
    ),
    out_shape=jax.ShapeDtypeStruct((1, N_ROWS), jnp.float32),
)


def kernel(state, index):
    idx = jnp.asarray(index, jnp.int32).reshape(1)
    return _gather_row(idx, state.T).reshape(N_ROWS)


# speedup vs baseline: 1.7685x; 1.0177x over previous
import jax
import jax.numpy as jnp
from jax.experimental import pallas as pl
from jax.experimental.pallas import tpu as pltpu

N_ROWS = 16384
N_COLS = 16320


def _gather_row_body(idx_ref, hbm_ref, out_ref, sem):
    row = idx_ref[0]
    copy = pltpu.make_async_copy(hbm_ref.at[pl.ds(row, 1), :], out_ref, sem)
    copy.start()
    copy.wait()


_gather_row = pl.pallas_call(
    _gather_row_body,
    grid_spec=pltpu.PrefetchScalarGridSpec(
        num_scalar_prefetch=1,
        grid=(1,),
        in_specs=[pl.BlockSpec(memory_space=pl.ANY)],
        out_specs=pl.BlockSpec((1, N_ROWS), lambda i, idx: (0, 0)),
        scratch_shapes=[pltpu.SemaphoreType.DMA],
    ),
    out_shape=jax.ShapeDtypeStruct((1, N_ROWS), jnp.float32),
)


def kernel(state, index):
    idx = jnp.asarray(index, jnp.int32).reshape(1)
    return _gather_row(idx, state.T).reshape(N_ROWS)
